# baseline (speedup 1.0000x reference)
import functools

import jax
import jax.numpy as jnp
from jax.experimental import pallas as pl
from jax.experimental.pallas import tpu as pltpu
from jax.experimental.pallas import tpu_sc as plsc

_WINDOW = 128


def _sc_gather_sum(word_emb, time_emb, ids_flat, tg_flat):
    n = ids_flat.shape[1]
    h = word_emb.shape[1]
    mesh = plsc.VectorSubcoreMesh(core_axis_name="c", subcore_axis_name="s")

    @functools.partial(
        pl.kernel,
        mesh=mesh,
        out_type=jax.ShapeDtypeStruct((n, h), jnp.float32),
        scratch_types=[
            pltpu.VMEM((_WINDOW, h), jnp.float32),
            pltpu.SemaphoreType.DMA,
            pltpu.SemaphoreType.DMA,
        ],
    )
    def k(word_hbm, time_hbm, ids_hbm, tg_hbm, o_hbm, tmp_v, sem_w, sem_t):
        def body(ids_v, tg_v, o_v):
            cw = pltpu.async_copy(word_hbm.at[ids_v.at[0]], o_v, sem_w)
            ct = pltpu.async_copy(time_hbm.at[tg_v.at[0]], tmp_v, sem_t)
            cw.wait()
            ct.wait()

            @pl.loop(0, _WINDOW)
            def _(r):
                for c in range(0, h, 16):
                    slc = (pl.ds(r, 1), pl.ds(c, 16))
                    o_v.at[*slc][...] = o_v.at[*slc][...] + tmp_v.at[*slc][...]

        pltpu.emit_pipeline(
            body,
            grid=(n // _WINDOW,),
            in_specs=[
                pl.BlockSpec((1, _WINDOW), lambda i: (0, i)),
                pl.BlockSpec((1, _WINDOW), lambda i: (0, i)),
            ],
            out_specs=[
                pl.BlockSpec((_WINDOW, h), lambda i: (i, 0)),
            ],
            core_axis_name=("c", "s"),
            dimension_semantics=(pltpu.PARALLEL,),
        )(ids_hbm, tg_hbm, o_hbm)

    return k(word_emb, time_emb, ids_flat, tg_flat)


def _tc_body(sum_ref, ids_ref, tt_ref, pos_ref, type_ref, w0_ref,
             gamma_ref, beta_ref, out_ref):
    ids = ids_ref[...]
    tt = tt_ref[...]
    pad = (ids[..., None] == 0).astype(jnp.float32)
    x = sum_ref[...] - w0_ref[...] * pad
    x = x + pos_ref[...][None]
    x = x + jnp.where(tt[..., None] == 0, type_ref[0], type_ref[1])
    mean = jnp.mean(x, axis=-1, keepdims=True)
    c = x - mean
    var = jnp.mean(c * c, axis=-1, keepdims=True)
    y = c * jax.lax.rsqrt(var + 1e-12)
    out_ref[...] = y * gamma_ref[...] + beta_ref[...]


def _tc_ln(sum_g, input_ids, token_type_ids, pos_s, type_emb, w0,
           gamma, beta, block_b=32):
    b, s, h = sum_g.shape
    return pl.pallas_call(
        _tc_body,
        grid=(b // block_b,),
        in_specs=[
            pl.BlockSpec((block_b, s, h), lambda i: (i, 0, 0)),
            pl.BlockSpec((block_b, s), lambda i: (i, 0)),
            pl.BlockSpec((block_b, s), lambda i: (i, 0)),
            pl.BlockSpec((s, h), lambda i: (0, 0)),
            pl.BlockSpec(type_emb.shape, lambda i: (0, 0)),
            pl.BlockSpec((1, h), lambda i: (0, 0)),
            pl.BlockSpec((1, h), lambda i: (0, 0)),
            pl.BlockSpec((1, h), lambda i: (0, 0)),
        ],
        out_specs=pl.BlockSpec((block_b, s, h), lambda i: (i, 0, 0)),
        out_shape=jax.ShapeDtypeStruct((b, s, h), jnp.float32),
    )(sum_g, input_ids, token_type_ids, pos_s, type_emb, w0, gamma, beta)


def kernel(input_ids, token_type_ids, time_gaps, word_emb, pos_emb, type_emb,
           time_emb, gamma, beta):
    b, s = input_ids.shape
    h = word_emb.shape[1]
    n = b * s
    sum_g = _sc_gather_sum(word_emb, time_emb,
                           input_ids.reshape(1, n), time_gaps.reshape(1, n))
    return _tc_ln(
        sum_g.reshape(b, s, h),
        input_ids,
        token_type_ids,
        pos_emb[:s],
        type_emb,
        word_emb[0:1],
        gamma.reshape(1, h),
        beta.reshape(1, h),
    )


# speedup vs baseline: 1.6041x; 1.6041x over previous
import functools

import jax
import jax.numpy as jnp
from jax.experimental import pallas as pl
from jax.experimental.pallas import tpu as pltpu
from jax.experimental.pallas import tpu_sc as plsc

_WINDOW = 128


def _sc_gather(word_emb, time_emb_bf16, ids_flat, tg_flat):
    n = ids_flat.shape[1]
    h = word_emb.shape[1]
    mesh = plsc.VectorSubcoreMesh(core_axis_name="c", subcore_axis_name="s")

    @functools.partial(
        pl.kernel,
        mesh=mesh,
        out_type=[
            jax.ShapeDtypeStruct((n, h), jnp.float32),
            jax.ShapeDtypeStruct((n, h), jnp.float32),
        ],
        scratch_types=[
            pltpu.SemaphoreType.DMA,
            pltpu.SemaphoreType.DMA,
        ],
    )
    def k(word_hbm, time_hbm, ids_hbm, tg_hbm, o_word_hbm, o_time_hbm,
          sem_w, sem_t):
        def body(ids_v, tg_v, o_word_v, o_time_v):
            cw = pltpu.async_copy(word_hbm.at[ids_v.at[0]], o_word_v, sem_w)
            ct = pltpu.async_copy(time_hbm.at[tg_v.at[0]], o_time_v, sem_t)
            cw.wait()
            ct.wait()

        pltpu.emit_pipeline(
            body,
            grid=(n // _WINDOW,),
            in_specs=[
                pl.BlockSpec((1, _WINDOW), lambda i: (0, i)),
                pl.BlockSpec((1, _WINDOW), lambda i: (0, i)),
            ],
            out_specs=[
                pl.BlockSpec((_WINDOW, h), lambda i: (i, 0)),
                pl.BlockSpec((_WINDOW, h), lambda i: (i, 0)),
            ],
            core_axis_name=("c", "s"),
            dimension_semantics=(pltpu.PARALLEL,),
        )(ids_hbm, tg_hbm, o_word_hbm, o_time_hbm)

    return k(word_emb, time_emb_bf16, ids_flat, tg_flat)


def _tc_body(word_ref, time_ref, ids_ref, tt_ref, pos_ref, type_ref,
             gamma_ref, beta_ref, out_ref):
    ids = ids_ref[...]
    tt = tt_ref[...]
    x = jnp.where(ids[..., None] != 0, word_ref[...], 0.0)
    x = x + time_ref[...]
    x = x + pos_ref[...][None]
    x = x + jnp.where(tt[..., None] == 0, type_ref[0], type_ref[1])
    mean = jnp.mean(x, axis=-1, keepdims=True)
    c = x - mean
    var = jnp.mean(c * c, axis=-1, keepdims=True)
    y = c * jax.lax.rsqrt(var + 1e-12)
    out_ref[...] = y * gamma_ref[...] + beta_ref[...]


def _tc_ln(word_g, time_g, input_ids, token_type_ids, pos_s, type_emb,
           gamma, beta, block_b=32):
    b, s, h = word_g.shape
    return pl.pallas_call(
        _tc_body,
        grid=(b // block_b,),
        in_specs=[
            pl.BlockSpec((block_b, s, h), lambda i: (i, 0, 0)),
            pl.BlockSpec((block_b, s, h), lambda i: (i, 0, 0)),
            pl.BlockSpec((block_b, s), lambda i: (i, 0)),
            pl.BlockSpec((block_b, s), lambda i: (i, 0)),
            pl.BlockSpec((s, h), lambda i: (0, 0)),
            pl.BlockSpec(type_emb.shape, lambda i: (0, 0)),
            pl.BlockSpec((1, h), lambda i: (0, 0)),
            pl.BlockSpec((1, h), lambda i: (0, 0)),
        ],
        out_specs=pl.BlockSpec((block_b, s, h), lambda i: (i, 0, 0)),
        out_shape=jax.ShapeDtypeStruct((b, s, h), jnp.float32),
    )(word_g, time_g, input_ids, token_type_ids, pos_s, type_emb,
      gamma, beta)


def kernel(input_ids, token_type_ids, time_gaps, word_emb, pos_emb, type_emb,
           time_emb, gamma, beta):
    b, s = input_ids.shape
    h = word_emb.shape[1]
    n = b * s
    word_g, time_g = _sc_gather(
        word_emb, time_emb,
        input_ids.reshape(1, n), time_gaps.reshape(1, n))
    return _tc_ln(
        word_g.reshape(b, s, h),
        time_g.reshape(b, s, h),
        input_ids,
        token_type_ids,
        pos_emb[:s],
        type_emb,
        gamma.reshape(1, h),
        beta.reshape(1, h),
    )


# speedup vs baseline: 1.6492x; 1.0281x over previous
import functools

import jax
import jax.numpy as jnp
from jax.experimental import pallas as pl
from jax.experimental.pallas import tpu as pltpu
from jax.experimental.pallas import tpu_sc as plsc

_WINDOW = 128


def _sc_gather(word_emb, time_emb_bf16, ids_flat, tg_flat):
    n = ids_flat.shape[1]
    h = word_emb.shape[1]
    mesh = plsc.VectorSubcoreMesh(core_axis_name="c", subcore_axis_name="s")

    @functools.partial(
        pl.kernel,
        mesh=mesh,
        out_type=[
            jax.ShapeDtypeStruct((n, h), jnp.float32),
            jax.ShapeDtypeStruct((n, h), jnp.float32),
        ],
        scratch_types=[
            pltpu.SemaphoreType.DMA,
            pltpu.SemaphoreType.DMA,
        ],
    )
    def k(word_hbm, time_hbm, ids_hbm, tg_hbm, o_word_hbm, o_time_hbm,
          sem_w, sem_t):
        def body(ids_v, tg_v, o_word_v, o_time_v):
            cw = pltpu.async_copy(word_hbm.at[ids_v.at[0]], o_word_v, sem_w)
            ct = pltpu.async_copy(time_hbm.at[tg_v.at[0]], o_time_v, sem_t)
            cw.wait()
            ct.wait()

        pltpu.emit_pipeline(
            body,
            grid=(n // _WINDOW,),
            in_specs=[
                pl.BlockSpec((1, _WINDOW), lambda i: (0, i)),
                pl.BlockSpec((1, _WINDOW), lambda i: (0, i)),
            ],
            out_specs=[
                pl.BlockSpec((_WINDOW, h), lambda i: (i, 0)),
                pl.BlockSpec((_WINDOW, h), lambda i: (i, 0)),
            ],
            core_axis_name=("c", "s"),
            dimension_semantics=(pltpu.PARALLEL,),
        )(ids_hbm, tg_hbm, o_word_hbm, o_time_hbm)

    return k(word_emb, time_emb_bf16, ids_flat, tg_flat)


def _tc_body(word_ref, time_ref, ids_ref, tt_ref, pos_ref, type_ref,
             gamma_ref, beta_ref, out_ref):
    ids = ids_ref[...]
    tt = tt_ref[...]
    x = jnp.where(ids[..., None] != 0, word_ref[...], 0.0)
    x = x + time_ref[...]
    x = x + pos_ref[...][None]
    x = x + jnp.where(tt[..., None] == 0, type_ref[0], type_ref[1])
    mean = jnp.mean(x, axis=-1, keepdims=True)
    c = x - mean
    var = jnp.mean(c * c, axis=-1, keepdims=True)
    y = c * jax.lax.rsqrt(var + 1e-12)
    out_ref[...] = y * gamma_ref[...] + beta_ref[...]


def _tc_ln(word_g, time_g, input_ids, token_type_ids, pos_s, type_emb,
           gamma, beta, block_b=64):
    b, s, h = word_g.shape
    return pl.pallas_call(
        _tc_body,
        grid=(b // block_b,),
        in_specs=[
            pl.BlockSpec((block_b, s, h), lambda i: (i, 0, 0)),
            pl.BlockSpec((block_b, s, h), lambda i: (i, 0, 0)),
            pl.BlockSpec((block_b, s), lambda i: (i, 0)),
            pl.BlockSpec((block_b, s), lambda i: (i, 0)),
            pl.BlockSpec((s, h), lambda i: (0, 0)),
            pl.BlockSpec(type_emb.shape, lambda i: (0, 0)),
            pl.BlockSpec((1, h), lambda i: (0, 0)),
            pl.BlockSpec((1, h), lambda i: (0, 0)),
        ],
        out_specs=pl.BlockSpec((block_b, s, h), lambda i: (i, 0, 0)),
        out_shape=jax.ShapeDtypeStruct((b, s, h), jnp.float32),
    )(word_g, time_g, input_ids, token_type_ids, pos_s, type_emb,
      gamma, beta)


def kernel(input_ids, token_type_ids, time_gaps, word_emb, pos_emb, type_emb,
           time_emb, gamma, beta):
    b, s = input_ids.shape
    h = word_emb.shape[1]
    n = b * s
    word_g, time_g = _sc_gather(
        word_emb, time_emb,
        input_ids.reshape(1, n), time_gaps.reshape(1, n))
    return _tc_ln(
        word_g.reshape(b, s, h),
        time_g.reshape(b, s, h),
        input_ids,
        token_type_ids,
        pos_emb[:s],
        type_emb,
        gamma.reshape(1, h),
        beta.reshape(1, h),
    )
